# baseline (speedup 1.0000x reference)
import functools
import numpy as np
import jax
import jax.numpy as jnp
from jax import lax
from jax.experimental import pallas as pl
from jax.experimental.pallas import tpu as pltpu
from jax.experimental.pallas import tpu_sc as plsc

F32 = jnp.float32
B, N, D = 256, 1024, 96
H, HD = 8, 12
TOPK = 512
C96 = np.float32(0.010416667)
C24 = np.float32(1.0 / 24.0)
C12 = np.float32(1.0 / 12.0)
EPS = np.float32(1e-5)


def _erfc(x):
    one = np.float32(1.0)
    ax = jnp.abs(x)
    x2 = x * x
    p = x2 * np.float32(7.85386146e-05)
    for c in (-0.000801019371, 0.00518832775, -0.0268538129, 0.112835854,
              -0.37612626):
        p = (p + np.float32(c)) * x2
    p = p + np.float32(1.12837911)
    res_a = one - x * p
    nx2 = -x2
    ex = jnp.exp(nx2)
    base = ex * (one / ax)
    w = one / x2
    q = w * np.float32(0.0232682)
    for c in (-0.138703942, 0.368742466, -0.582473278, 0.621000469,
              -0.494451523, 0.340488, -0.274112701):
        q = (q + np.float32(c)) * w
    q = q + np.float32(0.563825965)
    r = w * np.float32(-10.477664)
    for c in (12.9772, -7.49551868, 2.92101908, -1.01526523, 0.42184633,
              -0.282076746):
        r = (r + np.float32(c)) * w
    r = r + np.float32(0.564189494)
    poly = jnp.where(ax < np.float32(2.0), q, r)
    val = base * poly
    val = jnp.where(nx2 < np.float32(-88.7228394), np.float32(0.0), val)
    val = jnp.where(x < np.float32(0.0), np.float32(2.0) - val, val)
    return jnp.where(ax < one, res_a, val)


def _gelu(v):
    return (v * np.float32(0.5)) * _erfc((-v) * np.float32(0.707106769))


def _zsum(e):
    acc = e[:, 0:128]
    for i in range(1, 8):
        acc = acc + e[:, 128 * i:128 * (i + 1)]
    s = acc[:, 0:8]
    for i in range(1, 16):
        s = s + acc[:, 8 * i:8 * (i + 1)]
    s = s[:, 0:4] + s[:, 4:8]
    s = s[:, 0:2] + s[:, 2:4]
    return s[:, 0:1] + s[:, 1:2]


def _fold8(a):
    z = a[0:4] + a[4:8]
    z = z[0:2] + z[2:4]
    return z[0:1] + z[1:2]


def _sum96(x):
    ws = []
    for c in range(3):
        a = x[32 * c:32 * c + 8]
        for r in range(1, 4):
            a = a + x[32 * c + 8 * r:32 * c + 8 * r + 8]
        ws.append(_fold8(a))
    return (ws[0] + ws[1]) + ws[2]


def _sum24(x):
    a = (x[0:8] + x[8:16]) + x[16:24]
    return _fold8(a)


def _sum24_var(x):
    s0 = _fold8(x[0:8])
    s1 = _fold8(x[8:16])
    s2 = _fold8(x[16:24])
    return (s0 + s1) + s2


def _sum12(x):
    upper = x[0:4] + x[8:12]
    z = upper + x[4:8]
    z = z[0:2] + z[2:4]
    return z[0:1] + z[1:2]


def _score_kernel(pt_ref, lng_ref, lnb_ref, W1_ref, b1_ref, l1g_ref, l1b_ref,
                  W2_ref, b2_ref, l2g_ref, l2b_ref, W3_ref, b3_ref, Wf_ref,
                  bf_ref, aw_out):
    x = pt_ref[0]
    dot = functools.partial(lax.dot_general, precision='default',
                            preferred_element_type=F32)
    mu = _sum96(x) * C96
    xc = x - mu
    var = _sum96(xc * xc) * C96
    xn = xc / jnp.sqrt(var + EPS) * lng_ref[:] + lnb_ref[:]
    cols = []
    for h in range(H):
        a = dot(W1_ref[h], xn, (((0,), (0,)), ((), ()))) + b1_ref[h]
        mu1 = _sum24(a) * C24
        ac = a - mu1
        v1 = _sum24_var(ac * ac) * C24
        a = _gelu(ac / jnp.sqrt(v1 + EPS) * l1g_ref[h] + l1b_ref[h])
        c = dot(W2_ref[h], a, (((0,), (0,)), ((), ()))) + b2_ref[h]
        mu2 = _sum12(c) * C12
        cc = c - mu2
        v2 = _sum12(cc * cc) * C12
        c = _gelu(cc / jnp.sqrt(v2 + EPS) * l2g_ref[h] + l2b_ref[h])
        cols.append(dot(W3_ref[h], c, (((0,), (0,)), ((), ()))) + b3_ref[h])
    stacked = jnp.concatenate(cols, axis=0)
    fl = dot(Wf_ref[:], stacked, (((0,), (0,)), ((), ()))) + bf_ref[:]
    fused = jax.nn.sigmoid(fl)
    m = jnp.max(fused, axis=1, keepdims=True)
    e = jnp.exp(fused - m)
    aw_out[0] = e / _zsum(e)


def _scores(patches_t, ln_g, ln_b, W1, b1, ln1_g, ln1_b, W2, b2,
            ln2_g, ln2_b, W3, b3, Wf, bf):
    full = lambda s: pl.BlockSpec(s, lambda i: (0,) * len(s))
    blk = lambda s: pl.BlockSpec(s, lambda i: (i,) + (0,) * (len(s) - 1))
    in_specs = [
        blk((1, D, N)),
        full((D, 1)), full((D, 1)),
        full((H, D, 2 * HD)), full((H, 2 * HD, 1)), full((H, 2 * HD, 1)),
        full((H, 2 * HD, 1)),
        full((H, 2 * HD, HD)), full((H, HD, 1)), full((H, HD, 1)),
        full((H, HD, 1)),
        full((H, HD, 1)), full((H, 1, 1)), full((H, 1)), full((1, 1)),
    ]
    return pl.pallas_call(
        _score_kernel, grid=(B,),
        in_specs=in_specs,
        out_specs=blk((1, 1, N)),
        out_shape=jax.ShapeDtypeStruct((B, 1, N), F32),
    )(patches_t, ln_g.reshape(D, 1), ln_b.reshape(D, 1),
      W1, b1.reshape(H, 2 * HD, 1), ln1_g.reshape(H, 2 * HD, 1),
      ln1_b.reshape(H, 2 * HD, 1),
      W2, b2.reshape(H, HD, 1), ln2_g.reshape(H, HD, 1),
      ln2_b.reshape(H, HD, 1),
      W3, b3.reshape(H, 1, 1), Wf, bf.reshape(1, 1))


def _sort_kernel(aw_ref, idx_out):
    keys = aw_ref[:]
    row = lax.broadcasted_iota(jnp.int32, (N, B), 0)
    col = lax.broadcasted_iota(jnp.int32, (N, B), 1)
    idx = col * N + row
    size = 2
    while size <= N:
        d = size // 2
        while d >= 1:
            up_k = pltpu.roll(keys, N - d, 0)
            dn_k = pltpu.roll(keys, d, 0)
            up_i = pltpu.roll(idx, N - d, 0)
            dn_i = pltpu.roll(idx, d, 0)
            is_lower = (row & d) == 0
            pk = jnp.where(is_lower, up_k, dn_k)
            pi = jnp.where(is_lower, up_i, dn_i)
            wins = (keys > pk) | ((keys == pk) & (idx < pi))
            desc = (row & size) == 0
            keep = wins ^ is_lower ^ desc
            keys = jnp.where(keep, keys, pk)
            idx = jnp.where(keep, idx, pi)
            d //= 2
        size *= 2
    idx_out[:] = idx[:TOPK, :]


def _sort(aw_t):
    return pl.pallas_call(
        _sort_kernel,
        out_shape=jax.ShapeDtypeStruct((TOPK, B), jnp.int32),
    )(aw_t)


_CHUNK = 128


def _make_gather():
    info = plsc.get_sparse_core_info()
    nw = info.num_cores * info.num_subcores
    rows_per_w = (B * TOPK) // nw
    n_chunks = rows_per_w // _CHUNK
    mesh = plsc.VectorSubcoreMesh(core_axis_name="c", subcore_axis_name="s")

    @functools.partial(
        pl.kernel, mesh=mesh,
        compiler_params=pltpu.CompilerParams(use_tc_tiling_on_sc=False),
        out_type=jax.ShapeDtypeStruct((B * TOPK, D), F32),
        scratch_types=[
            pltpu.VMEM((rows_per_w,), jnp.int32),
            pltpu.VMEM((_CHUNK, D), F32),
            pltpu.VMEM((_CHUNK, D), F32),
            pltpu.SemaphoreType.DMA,
            pltpu.SemaphoreType.DMA,
        ],
    )
    def gather(flat_hbm, idx_hbm, out_hbm, idx_v, buf0, buf1, sem0, sem1):
        wid = lax.axis_index("s") * info.num_cores + lax.axis_index("c")
        base = wid * rows_per_w
        pltpu.sync_copy(idx_hbm.at[pl.ds(base, rows_per_w)], idx_v)
        bufs = (buf0, buf1)
        sems = (sem0, sem1)
        cps = [None, None]

        def start(j):
            p = j % 2
            cps[p] = pltpu.make_async_copy(
                flat_hbm.at[idx_v.at[pl.ds(j * _CHUNK, _CHUNK)]],
                bufs[p], sems[p])
            cps[p].start()

        start(0)
        for j in range(n_chunks):
            p = j % 2
            if j + 1 < n_chunks:
                start(j + 1)
            cps[p].wait()
            pltpu.sync_copy(bufs[p],
                            out_hbm.at[pl.ds(base + j * _CHUNK, _CHUNK)])

    return gather


def kernel(patches, ln_g, ln_b, W1, b1, ln1_g, ln1_b, W2, b2, ln2_g, ln2_b,
           W3, b3, Wf, bf):
    pt = jnp.transpose(patches, (0, 2, 1))
    aw3 = _scores(pt, ln_g, ln_b, W1, b1, ln1_g, ln1_b, W2, b2,
                  ln2_g, ln2_b, W3, b3, Wf, bf)
    aw = aw3.reshape(B, N)
    idx_t = _sort(jnp.transpose(aw))
    gidx = jnp.transpose(idx_t).reshape(B * TOPK)
    flat = patches.reshape(B * N, D)
    sel = _make_gather()(flat, gidx)
    return sel.reshape(B, TOPK, D), aw


# speedup vs baseline: 1.3572x; 1.3572x over previous
import functools
import numpy as np
import jax
import jax.numpy as jnp
from jax import lax
from jax.experimental import pallas as pl
from jax.experimental.pallas import tpu as pltpu
from jax.experimental.pallas import tpu_sc as plsc

F32 = jnp.float32
B, N, D = 256, 1024, 96
H, HD = 8, 12
TOPK = 512
C96 = np.float32(0.010416667)
C24 = np.float32(1.0 / 24.0)
C12 = np.float32(1.0 / 12.0)
EPS = np.float32(1e-5)


def _erfc(x):
    one = np.float32(1.0)
    ax = jnp.abs(x)
    x2 = x * x
    p = x2 * np.float32(7.85386146e-05)
    for c in (-0.000801019371, 0.00518832775, -0.0268538129, 0.112835854,
              -0.37612626):
        p = (p + np.float32(c)) * x2
    p = p + np.float32(1.12837911)
    res_a = one - x * p
    nx2 = -x2
    ex = jnp.exp(nx2)
    base = ex * (one / ax)
    w = one / x2
    q = w * np.float32(0.0232682)
    for c in (-0.138703942, 0.368742466, -0.582473278, 0.621000469,
              -0.494451523, 0.340488, -0.274112701):
        q = (q + np.float32(c)) * w
    q = q + np.float32(0.563825965)
    r = w * np.float32(-10.477664)
    for c in (12.9772, -7.49551868, 2.92101908, -1.01526523, 0.42184633,
              -0.282076746):
        r = (r + np.float32(c)) * w
    r = r + np.float32(0.564189494)
    poly = jnp.where(ax < np.float32(2.0), q, r)
    val = base * poly
    val = jnp.where(nx2 < np.float32(-88.7228394), np.float32(0.0), val)
    val = jnp.where(x < np.float32(0.0), np.float32(2.0) - val, val)
    return jnp.where(ax < one, res_a, val)


def _gelu(v):
    return (v * np.float32(0.5)) * _erfc((-v) * np.float32(0.707106769))


def _zsum(e):
    acc = e[:, 0:128]
    for i in range(1, 8):
        acc = acc + e[:, 128 * i:128 * (i + 1)]
    s = acc[:, 0:8]
    for i in range(1, 16):
        s = s + acc[:, 8 * i:8 * (i + 1)]
    s = s[:, 0:4] + s[:, 4:8]
    s = s[:, 0:2] + s[:, 2:4]
    return s[:, 0:1] + s[:, 1:2]


def _fold8(a):
    z = a[0:4] + a[4:8]
    z = z[0:2] + z[2:4]
    return z[0:1] + z[1:2]


def _sum96(x):
    ws = []
    for c in range(3):
        a = x[32 * c:32 * c + 8]
        for r in range(1, 4):
            a = a + x[32 * c + 8 * r:32 * c + 8 * r + 8]
        ws.append(_fold8(a))
    return (ws[0] + ws[1]) + ws[2]


def _sum24(x):
    a = (x[0:8] + x[8:16]) + x[16:24]
    return _fold8(a)


def _sum24_var(x):
    s0 = _fold8(x[0:8])
    s1 = _fold8(x[8:16])
    s2 = _fold8(x[16:24])
    return (s0 + s1) + s2


def _sum12(x):
    upper = x[0:4] + x[8:12]
    z = upper + x[4:8]
    z = z[0:2] + z[2:4]
    return z[0:1] + z[1:2]


_RPS = 8


def _score_kernel(pt_ref, lng_ref, lnb_ref, W1_ref, b1_ref, l1g_ref, l1b_ref,
                  W2_ref, b2_ref, l2g_ref, l2b_ref, W3_ref, b3_ref, Wf_ref,
                  bf_ref, aw_out):
    x = jnp.concatenate([pt_ref[r] for r in range(_RPS)], axis=1)
    dot = functools.partial(lax.dot_general, precision='default',
                            preferred_element_type=F32)
    mu = _sum96(x) * C96
    xc = x - mu
    var = _sum96(xc * xc) * C96
    xn = xc / jnp.sqrt(var + EPS) * lng_ref[:] + lnb_ref[:]
    cols = []
    for h in range(H):
        a = dot(W1_ref[h], xn, (((0,), (0,)), ((), ()))) + b1_ref[h]
        mu1 = _sum24(a) * C24
        ac = a - mu1
        v1 = _sum24_var(ac * ac) * C24
        a = _gelu(ac / jnp.sqrt(v1 + EPS) * l1g_ref[h] + l1b_ref[h])
        c = dot(W2_ref[h], a, (((0,), (0,)), ((), ()))) + b2_ref[h]
        mu2 = _sum12(c) * C12
        cc = c - mu2
        v2 = _sum12(cc * cc) * C12
        c = _gelu(cc / jnp.sqrt(v2 + EPS) * l2g_ref[h] + l2b_ref[h])
        cols.append(dot(W3_ref[h], c, (((0,), (0,)), ((), ()))) + b3_ref[h])
    stacked = jnp.concatenate(cols, axis=0)
    fl = dot(Wf_ref[:], stacked, (((0,), (0,)), ((), ()))) + bf_ref[:]
    fused = jax.nn.sigmoid(fl)
    for r in range(_RPS):
        fr = fused[:, r * N:(r + 1) * N]
        m = jnp.max(fr, axis=1, keepdims=True)
        e = jnp.exp(fr - m)
        aw_out[r] = e / _zsum(e)


def _scores(patches_t, ln_g, ln_b, W1, b1, ln1_g, ln1_b, W2, b2,
            ln2_g, ln2_b, W3, b3, Wf, bf):
    full = lambda s: pl.BlockSpec(s, lambda i: (0,) * len(s))
    blk = lambda s: pl.BlockSpec(s, lambda i: (i,) + (0,) * (len(s) - 1))
    in_specs = [
        blk((_RPS, D, N)),
        full((D, 1)), full((D, 1)),
        full((H, D, 2 * HD)), full((H, 2 * HD, 1)), full((H, 2 * HD, 1)),
        full((H, 2 * HD, 1)),
        full((H, 2 * HD, HD)), full((H, HD, 1)), full((H, HD, 1)),
        full((H, HD, 1)),
        full((H, HD, 1)), full((H, 1, 1)), full((H, 1)), full((1, 1)),
    ]
    return pl.pallas_call(
        _score_kernel, grid=(B // _RPS,),
        in_specs=in_specs,
        out_specs=blk((_RPS, 1, N)),
        out_shape=jax.ShapeDtypeStruct((B, 1, N), F32),
    )(patches_t, ln_g.reshape(D, 1), ln_b.reshape(D, 1),
      W1, b1.reshape(H, 2 * HD, 1), ln1_g.reshape(H, 2 * HD, 1),
      ln1_b.reshape(H, 2 * HD, 1),
      W2, b2.reshape(H, HD, 1), ln2_g.reshape(H, HD, 1),
      ln2_b.reshape(H, HD, 1),
      W3, b3.reshape(H, 1, 1), Wf, bf.reshape(1, 1))


def _sort_kernel(aw_ref, idx_out):
    keys = aw_ref[:]
    row = lax.broadcasted_iota(jnp.int32, (N, B), 0)
    col = lax.broadcasted_iota(jnp.int32, (N, B), 1)
    idx = col * N + row
    size = 2
    while size <= N:
        d = size // 2
        while d >= 1:
            up_k = pltpu.roll(keys, N - d, 0)
            dn_k = pltpu.roll(keys, d, 0)
            up_i = pltpu.roll(idx, N - d, 0)
            dn_i = pltpu.roll(idx, d, 0)
            is_lower = (row & d) == 0
            pk = jnp.where(is_lower, up_k, dn_k)
            pi = jnp.where(is_lower, up_i, dn_i)
            wins = (keys > pk) | ((keys == pk) & (idx < pi))
            desc = (row & size) == 0
            keep = wins ^ is_lower ^ desc
            keys = jnp.where(keep, keys, pk)
            idx = jnp.where(keep, idx, pi)
            d //= 2
        size *= 2
    idx_out[:] = idx[:TOPK, :]


def _sort(aw_t):
    return pl.pallas_call(
        _sort_kernel,
        out_shape=jax.ShapeDtypeStruct((TOPK, B), jnp.int32),
    )(aw_t)


_CHUNK = 128


def _make_gather():
    info = plsc.get_sparse_core_info()
    nw = info.num_cores * info.num_subcores
    rows_per_w = (B * TOPK) // nw
    n_chunks = rows_per_w // _CHUNK
    mesh = plsc.VectorSubcoreMesh(core_axis_name="c", subcore_axis_name="s")

    @functools.partial(
        pl.kernel, mesh=mesh,
        compiler_params=pltpu.CompilerParams(use_tc_tiling_on_sc=False),
        out_type=jax.ShapeDtypeStruct((B * TOPK, D), F32),
        scratch_types=[
            pltpu.VMEM((rows_per_w,), jnp.int32),
            pltpu.VMEM((_CHUNK, D), F32),
            pltpu.VMEM((_CHUNK, D), F32),
            pltpu.SemaphoreType.DMA,
            pltpu.SemaphoreType.DMA,
        ],
    )
    def gather(flat_hbm, idx_hbm, out_hbm, idx_v, buf0, buf1, sem0, sem1):
        wid = lax.axis_index("s") * info.num_cores + lax.axis_index("c")
        base = wid * rows_per_w
        pltpu.sync_copy(idx_hbm.at[pl.ds(base, rows_per_w)], idx_v)
        bufs = (buf0, buf1)
        sems = (sem0, sem1)
        cps = [None, None]

        def start(j):
            p = j % 2
            cps[p] = pltpu.make_async_copy(
                flat_hbm.at[idx_v.at[pl.ds(j * _CHUNK, _CHUNK)]],
                bufs[p], sems[p])
            cps[p].start()

        start(0)
        for j in range(n_chunks):
            p = j % 2
            if j + 1 < n_chunks:
                start(j + 1)
            cps[p].wait()
            pltpu.sync_copy(bufs[p],
                            out_hbm.at[pl.ds(base + j * _CHUNK, _CHUNK)])

    return gather


def kernel(patches, ln_g, ln_b, W1, b1, ln1_g, ln1_b, W2, b2, ln2_g, ln2_b,
           W3, b3, Wf, bf):
    pt = jnp.transpose(patches, (0, 2, 1))
    aw3 = _scores(pt, ln_g, ln_b, W1, b1, ln1_g, ln1_b, W2, b2,
                  ln2_g, ln2_b, W3, b3, Wf, bf)
    aw = aw3.reshape(B, N)
    idx_t = _sort(jnp.transpose(aw))
    gidx = jnp.transpose(idx_t).reshape(B * TOPK)
    flat = patches.reshape(B * N, D)
    sel = _make_gather()(flat, gidx)
    return sel.reshape(B, TOPK, D), aw
